# Initial kernel scaffold; baseline (speedup 1.0000x reference)
import jax
import jax.numpy as jnp
from jax.experimental import pallas as pl


def kernel(inputs, word_table, pos_table):
    raise NotImplementedError("write your pallas kernel here")


# speedup vs baseline: 3.1097x; 3.1097x over previous
import functools

import jax
import jax.numpy as jnp
from jax import lax
from jax.experimental import pallas as pl
from jax.experimental.pallas import tpu as pltpu
from jax.experimental.pallas import tpu_sc as plsc

SEQ = 50
D = 128
BATCH = 4096
TOT = BATCH * SEQ
NC, NS = 2, 16
NW = NC * NS
PER_W = TOT // NW
CHUNK = 400
NCHUNK = PER_W // CHUNK
SUB = 80
NSUB = CHUNK // SUB
LP = SEQ * D // 16
ROWS_PER_P = CHUNK // SEQ

_mesh = plsc.VectorSubcoreMesh(
    core_axis_name="c", subcore_axis_name="s", num_cores=NC, num_subcores=NS
)


@functools.partial(
    pl.kernel,
    out_type=jax.ShapeDtypeStruct((TOT, D), jnp.float32),
    mesh=_mesh,
    scratch_types=[
        pltpu.VMEM((2, NSUB, SUB), jnp.int32),
        pltpu.VMEM((2, CHUNK, D), jnp.float32),
        pltpu.VMEM((SEQ * D,), jnp.float32),
        pltpu.SemaphoreType.DMA,
        pltpu.SemaphoreType.DMA,
        pltpu.SemaphoreType.DMA,
        pltpu.SemaphoreType.DMA,
    ],
)
def _sc_embed(idx_hbm, table_hbm, pos_hbm, out_hbm, idx_v, rows_v, pos_v,
              gsem0, gsem1, osem0, osem1):
    wid = lax.axis_index("s") * NC + lax.axis_index("c")
    base = wid * PER_W
    cbase = wid * NCHUNK
    gsems = (gsem0, gsem1)
    osems = (osem0, osem1)

    pltpu.sync_copy(pos_hbm, pos_v)

    def fire_gather(c, slot):
        pltpu.sync_copy(idx_hbm.at[cbase + c], idx_v.at[slot])
        return [
            pltpu.async_copy(
                table_hbm.at[idx_v.at[slot, j]],
                rows_v.at[slot, pl.ds(j * SUB, SUB)],
                gsems[slot],
            )
            for j in range(NSUB)
        ]

    def add_pos(slot):
        rows = rows_v.at[slot]

        def body(p, carry):
            pv = pos_v[pl.ds(p * 16, 16)]
            r = p // (D // 16)
            dc = (p % (D // 16)) * 16
            for k in range(ROWS_PER_P):
                plsc.addupdate(rows.at[k * SEQ + r, pl.ds(dc, 16)], pv)
            return carry

        lax.fori_loop(0, LP, body, 0, unroll=2)

    gh = {0: fire_gather(0, 0)}
    oh = {}
    for c in range(NCHUNK):
        slot = c % 2
        if c + 1 < NCHUNK:
            if c - 1 >= 0:
                oh[c - 1].wait()
            gh[c + 1] = fire_gather(c + 1, 1 - slot)
        for h in gh.pop(c):
            h.wait()
        add_pos(slot)
        oh[c] = pltpu.async_copy(
            rows_v.at[slot],
            out_hbm.at[pl.ds(base + c * CHUNK, CHUNK)],
            osems[slot],
        )
    oh[NCHUNK - 2].wait()
    oh[NCHUNK - 1].wait()


def kernel(inputs, word_table, pos_table):
    idx = jnp.reshape(inputs.astype(jnp.int32), (NW * NCHUNK, NSUB, SUB))
    pos_flat = jnp.reshape(pos_table, (SEQ * D,))
    out = _sc_embed(idx, word_table, pos_flat)
    return jnp.reshape(out, (BATCH, SEQ, D))


# speedup vs baseline: 3.1903x; 1.0259x over previous
import functools

import jax
import jax.numpy as jnp
from jax import lax
from jax.experimental import pallas as pl
from jax.experimental.pallas import tpu as pltpu
from jax.experimental.pallas import tpu_sc as plsc

SEQ = 50
D = 128
BATCH = 4096
TOT = BATCH * SEQ
NC, NS = 2, 16
NW = NC * NS
PER_W = TOT // NW
CHUNK = 400
NCHUNK = PER_W // CHUNK
SUB = 80
NSUB = CHUNK // SUB
LP = SEQ * D // 16
ROWS_PER_P = CHUNK // SEQ

_mesh = plsc.VectorSubcoreMesh(
    core_axis_name="c", subcore_axis_name="s", num_cores=NC, num_subcores=NS
)


@functools.partial(
    pl.kernel,
    out_type=jax.ShapeDtypeStruct((TOT, D), jnp.float32),
    mesh=_mesh,
    scratch_types=[
        pltpu.VMEM((NCHUNK, NSUB, SUB), jnp.int32),
        pltpu.VMEM((2, CHUNK, D), jnp.float32),
        pltpu.VMEM((SEQ * D,), jnp.float32),
        pltpu.SemaphoreType.DMA,
        pltpu.SemaphoreType.DMA,
        pltpu.SemaphoreType.DMA,
        pltpu.SemaphoreType.DMA,
    ],
)
def _sc_embed(idx_hbm, table_hbm, pos_hbm, out_hbm, idx_v, rows_v, pos_v,
              gsem0, gsem1, osem0, osem1):
    wid = lax.axis_index("s") * NC + lax.axis_index("c")
    base = wid * PER_W
    cbase = wid * NCHUNK
    gsems = (gsem0, gsem1)
    osems = (osem0, osem1)

    pltpu.sync_copy(pos_hbm, pos_v)
    pltpu.sync_copy(idx_hbm.at[pl.ds(cbase, NCHUNK)], idx_v)

    def fire_gather(c, slot):
        return [
            pltpu.async_copy(
                table_hbm.at[idx_v.at[c, j]],
                rows_v.at[slot, pl.ds(j * SUB, SUB)],
                gsems[slot],
            )
            for j in range(NSUB)
        ]

    def add_pos(slot):
        rows = rows_v.at[slot]

        def body(p, carry):
            pv = pos_v[pl.ds(p * 16, 16)]
            r = p // (D // 16)
            dc = (p % (D // 16)) * 16
            for k in range(ROWS_PER_P):
                plsc.addupdate(rows.at[k * SEQ + r, pl.ds(dc, 16)], pv)
            return carry

        lax.fori_loop(0, LP, body, 0, unroll=4)

    gh = {0: fire_gather(0, 0)}
    oh = {}
    for c in range(NCHUNK):
        slot = c % 2
        if c + 1 < NCHUNK:
            if c - 1 >= 0:
                oh[c - 1].wait()
            gh[c + 1] = fire_gather(c + 1, 1 - slot)
        for h in gh.pop(c):
            h.wait()
        add_pos(slot)
        oh[c] = pltpu.async_copy(
            rows_v.at[slot],
            out_hbm.at[pl.ds(base + c * CHUNK, CHUNK)],
            osems[slot],
        )
    oh[NCHUNK - 2].wait()
    oh[NCHUNK - 1].wait()


def kernel(inputs, word_table, pos_table):
    idx = jnp.reshape(inputs.astype(jnp.int32), (NW * NCHUNK, NSUB, SUB))
    pos_flat = jnp.reshape(pos_table, (SEQ * D,))
    out = _sc_embed(idx, word_table, pos_flat)
    return jnp.reshape(out, (BATCH, SEQ, D))


# speedup vs baseline: 5.4122x; 1.6964x over previous
import functools

import jax
import jax.numpy as jnp
from jax import lax
from jax.experimental import pallas as pl
from jax.experimental.pallas import tpu as pltpu
from jax.experimental.pallas import tpu_sc as plsc

SEQ = 50
D = 128
BATCH = 4096
TOT = BATCH * SEQ
NC, NS = 2, 16
NW = NC * NS
PER_W = TOT // NW
CHUNK = 400
NCHUNK = PER_W // CHUNK
SUB = 80
NSUB = CHUNK // SUB
LP = SEQ * D // 16
ROWS_PER_P = CHUNK // SEQ

_mesh = plsc.VectorSubcoreMesh(
    core_axis_name="c", subcore_axis_name="s", num_cores=NC, num_subcores=NS
)


@functools.partial(
    pl.kernel,
    out_type=jax.ShapeDtypeStruct((BATCH, SEQ, D), jnp.float32),
    mesh=_mesh,
    scratch_types=[
        pltpu.VMEM((NCHUNK, NSUB, SUB), jnp.int32),
        pltpu.VMEM((2, CHUNK, D), jnp.float32),
        pltpu.VMEM((SEQ * D,), jnp.float32),
        pltpu.SemaphoreType.DMA,
        pltpu.SemaphoreType.DMA,
        pltpu.SemaphoreType.DMA,
        pltpu.SemaphoreType.DMA,
    ],
)
def _sc_embed(idx_hbm, table_hbm, pos_hbm, out_hbm, idx_v, rows_v, pos_v,
              gsem0, gsem1, osem0, osem1):
    wid = lax.axis_index("s") * NC + lax.axis_index("c")
    base = wid * PER_W
    cbase = wid * NCHUNK
    gsems = (gsem0, gsem1)
    osems = (osem0, osem1)

    pltpu.sync_copy(pos_hbm, pos_v)
    pltpu.sync_copy(idx_hbm.at[pl.ds(cbase, NCHUNK)], idx_v)

    def fire_gather(c, slot):
        return [
            pltpu.async_copy(
                table_hbm.at[idx_v.at[c, j]],
                rows_v.at[slot, pl.ds(j * SUB, SUB)],
                gsems[slot],
            )
            for j in range(NSUB)
        ]

    def add_pos(slot):
        rows = rows_v.at[slot]

        def body(p, carry):
            pv = pos_v[pl.ds(p * 16, 16)]
            r = p // (D // 16)
            dc = (p % (D // 16)) * 16
            for k in range(ROWS_PER_P):
                plsc.addupdate(rows.at[k * SEQ + r, pl.ds(dc, 16)], pv)
            return carry

        lax.fori_loop(0, LP, body, 0, unroll=4)

    gh = {0: fire_gather(0, 0)}
    oh = {}
    for c in range(NCHUNK):
        slot = c % 2
        if c + 1 < NCHUNK:
            if c - 1 >= 0:
                for h in oh[c - 1]:
                    h.wait()
            gh[c + 1] = fire_gather(c + 1, 1 - slot)
        for h in gh.pop(c):
            h.wait()
        add_pos(slot)
        b0 = (base + c * CHUNK) // SEQ
        oh[c] = [
            pltpu.async_copy(
                rows_v.at[slot, pl.ds(k * SEQ, SEQ)],
                out_hbm.at[b0 + k],
                osems[slot],
            )
            for k in range(CHUNK // SEQ)
        ]
    for h in oh[NCHUNK - 2]:
        h.wait()
    for h in oh[NCHUNK - 1]:
        h.wait()


def kernel(inputs, word_table, pos_table):
    idx = jnp.reshape(inputs.astype(jnp.int32), (NW * NCHUNK, NSUB, SUB))
    pos_flat = jnp.reshape(pos_table, (SEQ * D,))
    return _sc_embed(idx, word_table, pos_flat)
